# baseline (speedup 1.0000x reference)
import jax
import jax.numpy as jnp
from jax.experimental import pallas as pl

B, N, T, H = 8, 325, 168, 64
N_BLK = 65


def _body(hour_ref, start_ref, end_ref, pos_ref, node_ref, out_ref):
    row = jax.lax.broadcasted_iota(jnp.int32, (24, 1), 0)
    table = (pos_ref[...]
             + jnp.where(row == 0, 1.0, 0.0) * start_ref[...][None, :]
             + jnp.where(row == 23, 1.0, 0.0) * end_ref[...][None, :])
    hour = hour_ref[0, 0, :]
    col = jax.lax.broadcasted_iota(jnp.int32, (T, 24), 1)
    onehot = (hour[:, None] == col).astype(jnp.float32)
    add = jnp.dot(onehot, table, preferred_element_type=jnp.float32)
    out_ref[...] = node_ref[...] + add[None, None, :, :]


def kernel(node_emb, hour_of_day, session_start, session_end, position_emb):
    hour3 = hour_of_day.astype(jnp.int32).reshape(B, 1, T)
    return pl.pallas_call(
        _body,
        grid=(B, N // N_BLK),
        in_specs=[
            pl.BlockSpec((1, 1, T), lambda b, n: (b, 0, 0)),
            pl.BlockSpec((H,), lambda b, n: (0,)),
            pl.BlockSpec((H,), lambda b, n: (0,)),
            pl.BlockSpec((24, H), lambda b, n: (0, 0)),
            pl.BlockSpec((1, N_BLK, T, H), lambda b, n: (b, n, 0, 0)),
        ],
        out_specs=pl.BlockSpec((1, N_BLK, T, H), lambda b, n: (b, n, 0, 0)),
        out_shape=jax.ShapeDtypeStruct((B, N, T, H), jnp.float32),
    )(hour3, session_start, session_end, position_emb, node_emb)


# speedup vs baseline: 1.6233x; 1.6233x over previous
import jax
import jax.numpy as jnp
from jax.experimental import pallas as pl

B, N, T, H = 8, 325, 168, 64
C = 2
CH = T * H // C


def _gather_body(hour_ref, start_ref, end_ref, pos_ref, out_ref):
    row = jax.lax.broadcasted_iota(jnp.int32, (24, 1), 0)
    table = (pos_ref[...]
             + jnp.where(row == 0, 1.0, 0.0) * start_ref[...][None, :]
             + jnp.where(row == 23, 1.0, 0.0) * end_ref[...][None, :])
    hour = hour_ref[0, 0, :]
    col = jax.lax.broadcasted_iota(jnp.int32, (T, 24), 1)
    onehot = (hour[:, None] == col).astype(jnp.float32)
    out_ref[0] = jnp.dot(onehot, table, preferred_element_type=jnp.float32)


def _add_body(node_ref, add_ref, out_ref):
    out_ref[...] = node_ref[...] + add_ref[...]


def kernel(node_emb, hour_of_day, session_start, session_end, position_emb):
    hour3 = hour_of_day.astype(jnp.int32).reshape(B, 1, T)
    add = pl.pallas_call(
        _gather_body,
        grid=(B,),
        in_specs=[
            pl.BlockSpec((1, 1, T), lambda b: (b, 0, 0)),
            pl.BlockSpec((H,), lambda b: (0,)),
            pl.BlockSpec((H,), lambda b: (0,)),
            pl.BlockSpec((24, H), lambda b: (0, 0)),
        ],
        out_specs=pl.BlockSpec((1, T, H), lambda b: (b, 0, 0)),
        out_shape=jax.ShapeDtypeStruct((B, T, H), jnp.float32),
    )(hour3, session_start, session_end, position_emb)

    node2 = node_emb.reshape(B, N, T * H)
    add2 = add.reshape(B, 1, T * H)
    out2 = pl.pallas_call(
        _add_body,
        grid=(B, C),
        in_specs=[
            pl.BlockSpec((1, N, CH), lambda b, c: (b, 0, c)),
            pl.BlockSpec((1, 1, CH), lambda b, c: (b, 0, c)),
        ],
        out_specs=pl.BlockSpec((1, N, CH), lambda b, c: (b, 0, c)),
        out_shape=jax.ShapeDtypeStruct((B, N, T * H), jnp.float32),
    )(node2, add2)
    return out2.reshape(B, N, T, H)
